# baseline (speedup 1.0000x reference)
import functools

import jax
import jax.numpy as jnp
from jax import lax
from jax.experimental import pallas as pl
from jax.experimental.pallas import tpu as pltpu
from jax.experimental.pallas import tpu_sc as plsc

_NC = 2
_NS = 16
_NW = _NC * _NS
_L = 16
_D = 64
_CHUNK = 128
_NBUF = 4


def _rsqrt16(x):
    i = plsc.bitcast(x, jnp.int32)
    y = plsc.bitcast(jnp.int32(0x5F3759DF) - (i >> 1), jnp.float32)
    for _ in range(2):
        y = y * (1.5 - 0.5 * x * y * y)
    return y


@functools.lru_cache(maxsize=None)
def _make_kernel(n_chunks, blocks_per_field, fields):
    assert n_chunks % _NBUF == 0 and n_chunks >= 2 * _NBUF
    mesh = plsc.VectorSubcoreMesh(core_axis_name="c", subcore_axis_name="s")
    ngrp = _CHUNK // _L

    @functools.partial(
        pl.kernel,
        mesh=mesh,
        compiler_params=pltpu.CompilerParams(
            needs_layout_passes=False, use_tc_tiling_on_sc=False
        ),
        out_type=jax.ShapeDtypeStruct(
            (fields, _D // 8, blocks_per_field, 8, _CHUNK), jnp.float32
        ),
        scratch_types=[
            pltpu.VMEM((n_chunks, _CHUNK), jnp.int32),
            *([pltpu.VMEM((_CHUNK, _D), jnp.float32)] * _NBUF),
            *([pltpu.VMEM((_D // 8, 8, _CHUNK), jnp.float32)] * _NBUF),
            *([pltpu.SemaphoreType.DMA] * (2 * _NBUF)),
        ],
    )
    def k(idx_hbm, tab_hbm, out_hbm, idx_v, *bufs_sems):
        bufs = bufs_sems[:_NBUF]
        tbufs = bufs_sems[_NBUF : 2 * _NBUF]
        gsems = bufs_sems[2 * _NBUF : 3 * _NBUF]
        osems = bufs_sems[3 * _NBUF :]
        cid = lax.axis_index("c")
        sid = lax.axis_index("s")
        wid = sid * _NC + cid
        pltpu.sync_copy(idx_hbm.at[wid], idx_v)
        lane = lax.iota(jnp.int32, _L)
        rowv = [lane + cg * _L for cg in range(ngrp)]
        perms = [lane ^ sh for sh in (8, 4, 2, 1)]

        def start_gather(g, j):
            pltpu.async_copy(tab_hbm.at[idx_v.at[g]], bufs[j], gsems[j])

        def wait_gather(j):
            pltpu.make_async_copy(tab_hbm.at[idx_v.at[0]], bufs[j], gsems[j]).wait()

        def start_out(h, j):
            beta = wid * n_chunks + h
            f = beta // blocks_per_field
            bt = beta % blocks_per_field
            pltpu.async_copy(tbufs[j], out_hbm.at[f, :, bt, :, :], osems[j])

        def wait_out(j):
            pltpu.make_async_copy(tbufs[j], out_hbm.at[0, :, 0, :, :], osems[j]).wait()

        def compute(j):
            buf = bufs[j]
            tbuf = tbufs[j]

            scales = []
            for cg in range(ngrp):
                def rows4(t, pack):
                    for u in range(4):
                        l = t * 4 + u
                        row = cg * _L + l
                        q0 = buf[row, pl.ds(0, _L)]
                        q1 = buf[row, pl.ds(_L, _L)]
                        q2 = buf[row, pl.ds(2 * _L, _L)]
                        q3 = buf[row, pl.ds(3 * _L, _L)]
                        s = q0 * q0 + q1 * q1 + q2 * q2 + q3 * q3
                        for p in perms:
                            s = s + jnp.take_along_axis(s, p, axis=0)
                        pack = jnp.where(lane == l, s, pack)
                    return pack

                tot = lax.fori_loop(0, _L // 4, rows4, jnp.zeros((_L,), jnp.float32))
                scales.append(jnp.where(tot > 1.0, _rsqrt16(tot), 1.0))

            def p2(t, carry):
                for u in range(4):
                    d = t * 4 + u
                    colv = (jnp.full((_L,), d, jnp.int32) + lane) & (_D - 1)
                    dtv = colv >> 3
                    drv = colv & 7
                    for cg in range(ngrp):
                        v = plsc.load_gather(buf, [rowv[cg], colv]) * scales[cg]
                        plsc.store_scatter(tbuf, [dtv, drv, rowv[cg]], v)
                return carry

            lax.fori_loop(0, _D // 4, p2, 0)

        for j in range(_NBUF - 1):
            start_gather(j, j)

        def outer(t, carry):
            for u in range(_NBUF):
                h = t * _NBUF + u
                j = u
                jj = (j + _NBUF - 1) % _NBUF
                wait_gather(j)
                compute(j)

                @pl.when(h >= 1)
                def _drain():
                    wait_out(jj)

                start_out(h, j)

                @pl.when(h + _NBUF - 1 < n_chunks)
                def _prefetch():
                    start_gather(h + _NBUF - 1, jj)

            return carry

        lax.fori_loop(0, n_chunks // _NBUF, outer, 0)
        wait_out((n_chunks - 1) % _NBUF)

    return k


def kernel(indices, node_emb):
    bsz, fields = indices.shape
    n_rows = bsz * fields
    assert n_rows % (_NW * _CHUNK) == 0, n_rows
    assert bsz % _CHUNK == 0
    n_chunks = n_rows // (_NW * _CHUNK)
    bpf = bsz // _CHUNK
    idx3 = indices.T.reshape(_NW, n_chunks, _CHUNK)
    out5 = _make_kernel(n_chunks, bpf, fields)(idx3, node_emb)
    return out5.transpose(2, 4, 0, 1, 3).reshape(bsz, fields, _D)


# speedup vs baseline: 1.0710x; 1.0710x over previous
import functools

import jax
import jax.numpy as jnp
from jax import lax
from jax.experimental import pallas as pl
from jax.experimental.pallas import tpu as pltpu
from jax.experimental.pallas import tpu_sc as plsc

_NC = 2
_NS = 16
_NW = _NC * _NS
_L = 16
_D = 64
_CHUNK = 128
_NBUF = 4


def _rsqrt16(x):
    i = plsc.bitcast(x, jnp.int32)
    y = plsc.bitcast(jnp.int32(0x5F3759DF) - (i >> 1), jnp.float32)
    for _ in range(2):
        y = y * (1.5 - 0.5 * x * y * y)
    return y


@functools.lru_cache(maxsize=None)
def _make_kernel(n_chunks, blocks_per_field, fields):
    assert n_chunks % _NBUF == 0 and n_chunks >= 2 * _NBUF
    mesh = plsc.VectorSubcoreMesh(core_axis_name="c", subcore_axis_name="s")
    ngrp = _CHUNK // _L

    @functools.partial(
        pl.kernel,
        mesh=mesh,
        compiler_params=pltpu.CompilerParams(
            needs_layout_passes=False, use_tc_tiling_on_sc=False
        ),
        out_type=jax.ShapeDtypeStruct(
            (fields, _D // 8, blocks_per_field, 8, _CHUNK), jnp.float32
        ),
        scratch_types=[
            pltpu.VMEM((n_chunks, _CHUNK), jnp.int32),
            *([pltpu.VMEM((_CHUNK, _D), jnp.float32)] * _NBUF),
            *([pltpu.VMEM((_D // 8, 8, _CHUNK), jnp.float32)] * _NBUF),
            *([pltpu.SemaphoreType.DMA] * (2 * _NBUF)),
        ],
    )
    def k(idx_hbm, tab_hbm, out_hbm, idx_v, *bufs_sems):
        bufs = bufs_sems[:_NBUF]
        tbufs = bufs_sems[_NBUF : 2 * _NBUF]
        gsems = bufs_sems[2 * _NBUF : 3 * _NBUF]
        osems = bufs_sems[3 * _NBUF :]
        cid = lax.axis_index("c")
        sid = lax.axis_index("s")
        wid = sid * _NC + cid
        pltpu.sync_copy(idx_hbm.at[wid], idx_v)
        lane = lax.iota(jnp.int32, _L)
        rowv = [lane + cg * _L for cg in range(ngrp)]
        perms = [lane ^ sh for sh in (8, 4, 2, 1)]

        def start_gather(g, j):
            pltpu.async_copy(tab_hbm.at[idx_v.at[g]], bufs[j], gsems[j])

        def wait_gather(j):
            pltpu.make_async_copy(tab_hbm.at[idx_v.at[0]], bufs[j], gsems[j]).wait()

        def start_out(h, j):
            beta = wid * n_chunks + h
            f = beta // blocks_per_field
            bt = beta % blocks_per_field
            pltpu.async_copy(tbufs[j], out_hbm.at[f, :, bt, :, :], osems[j])

        def wait_out(j):
            pltpu.make_async_copy(tbufs[j], out_hbm.at[0, :, 0, :, :], osems[j]).wait()

        def compute(j):
            buf = bufs[j]
            tbuf = tbufs[j]

            def p1(t, accs):
                accs = list(accs)
                for u in range(4):
                    d = t * 4 + u
                    colv = (jnp.full((_L,), d, jnp.int32) + lane) & (_D - 1)
                    for cg in range(ngrp):
                        v = plsc.load_gather(buf, [rowv[cg], colv])
                        accs[cg] = accs[cg] + v * v
                return tuple(accs)

            accs = lax.fori_loop(
                0,
                _D // 4,
                p1,
                tuple(jnp.zeros((_L,), jnp.float32) for _ in range(ngrp)),
            )
            scales = [jnp.where(a > 1.0, _rsqrt16(a), 1.0) for a in accs]

            def p2(t, carry):
                for u in range(4):
                    d = t * 4 + u
                    colv = (jnp.full((_L,), d, jnp.int32) + lane) & (_D - 1)
                    dtv = colv >> 3
                    drv = colv & 7
                    for cg in range(ngrp):
                        v = plsc.load_gather(buf, [rowv[cg], colv]) * scales[cg]
                        plsc.store_scatter(tbuf, [dtv, drv, rowv[cg]], v)
                return carry

            lax.fori_loop(0, _D // 4, p2, 0)

        for j in range(_NBUF - 1):
            start_gather(j, j)

        def outer(t, carry):
            for u in range(_NBUF):
                h = t * _NBUF + u
                j = u
                jj = (j + _NBUF - 1) % _NBUF
                wait_gather(j)
                compute(j)

                @pl.when(h >= 1)
                def _drain():
                    wait_out(jj)

                start_out(h, j)

                @pl.when(h + _NBUF - 1 < n_chunks)
                def _prefetch():
                    start_gather(h + _NBUF - 1, jj)

            return carry

        lax.fori_loop(0, n_chunks // _NBUF, outer, 0)
        wait_out((n_chunks - 1) % _NBUF)

    return k


def kernel(indices, node_emb):
    bsz, fields = indices.shape
    n_rows = bsz * fields
    assert n_rows % (_NW * _CHUNK) == 0, n_rows
    assert bsz % _CHUNK == 0
    n_chunks = n_rows // (_NW * _CHUNK)
    bpf = bsz // _CHUNK
    idx3 = indices.T.reshape(_NW, n_chunks, _CHUNK)
    out5 = _make_kernel(n_chunks, bpf, fields)(idx3, node_emb)
    return out5.transpose(2, 4, 0, 1, 3).reshape(bsz, fields, _D)


# speedup vs baseline: 1.3655x; 1.2749x over previous
import functools

import jax
import jax.numpy as jnp
from jax import lax
from jax.experimental import pallas as pl
from jax.experimental.pallas import tpu as pltpu
from jax.experimental.pallas import tpu_sc as plsc

_NC = 2
_NS = 16
_NW = _NC * _NS
_L = 16
_D = 64
_CHUNK = 128
_NBUF = 4


def _rsqrt16(x):
    i = plsc.bitcast(x, jnp.int32)
    y = plsc.bitcast(jnp.int32(0x5F3759DF) - (i >> 1), jnp.float32)
    for _ in range(2):
        y = y * (1.5 - 0.5 * x * y * y)
    return y


@functools.lru_cache(maxsize=None)
def _make_kernel(n_chunks, blocks_per_field, fields):
    assert n_chunks % _NBUF == 0 and n_chunks >= 2 * _NBUF
    mesh = plsc.VectorSubcoreMesh(core_axis_name="c", subcore_axis_name="s")
    ngrp = _CHUNK // _L

    @functools.partial(
        pl.kernel,
        mesh=mesh,
        compiler_params=pltpu.CompilerParams(
            needs_layout_passes=False, use_tc_tiling_on_sc=False
        ),
        out_type=jax.ShapeDtypeStruct(
            (fields, _D // 8, blocks_per_field, 8, _CHUNK), jnp.float32
        ),
        scratch_types=[
            pltpu.VMEM((n_chunks, _CHUNK), jnp.int32),
            *([pltpu.VMEM((_CHUNK, _D), jnp.float32)] * _NBUF),
            *([pltpu.VMEM((_D // 8, 8, _CHUNK), jnp.float32)] * _NBUF),
            *([pltpu.SemaphoreType.DMA] * (2 * _NBUF)),
        ],
    )
    def k(idx_hbm, tab_hbm, out_hbm, idx_v, *bufs_sems):
        bufs = bufs_sems[:_NBUF]
        tbufs = bufs_sems[_NBUF : 2 * _NBUF]
        gsems = bufs_sems[2 * _NBUF : 3 * _NBUF]
        osems = bufs_sems[3 * _NBUF :]
        cid = lax.axis_index("c")
        sid = lax.axis_index("s")
        wid = sid * _NC + cid
        pltpu.sync_copy(idx_hbm.at[wid], idx_v)
        lane = lax.iota(jnp.int32, _L)
        rowv = [lane + cg * _L for cg in range(ngrp)]
        perms = [lane ^ sh for sh in (8, 4, 2, 1)]

        def start_gather(g, j):
            pltpu.async_copy(tab_hbm.at[idx_v.at[g]], bufs[j], gsems[j])

        def wait_gather(j):
            pltpu.make_async_copy(tab_hbm.at[idx_v.at[0]], bufs[j], gsems[j]).wait()

        def start_out(h, j):
            beta = wid * n_chunks + h
            f = beta // blocks_per_field
            bt = beta % blocks_per_field
            pltpu.async_copy(tbufs[j], out_hbm.at[f, :, bt, :, :], osems[j])

        def wait_out(j):
            pltpu.make_async_copy(tbufs[j], out_hbm.at[0, :, 0, :, :], osems[j]).wait()

        def compute(j):
            buf = bufs[j]
            tbuf = tbufs[j]

            def p1(d, accs):
                colv = (jnp.full((_L,), d, jnp.int32) + lane) & (_D - 1)
                new = []
                for cg in range(ngrp):
                    v = plsc.load_gather(buf, [rowv[cg], colv])
                    new.append(accs[cg] + v * v)
                return tuple(new)

            accs = plsc.parallel_loop(
                0,
                _D,
                unroll=4,
                carry=tuple(jnp.zeros((_L,), jnp.float32) for _ in range(ngrp)),
            )(p1)
            scales = [jnp.where(a > 1.0, _rsqrt16(a), 1.0) for a in accs]

            def p2(d):
                colv = (jnp.full((_L,), d, jnp.int32) + lane) & (_D - 1)
                dtv = colv >> 3
                drv = colv & 7
                for cg in range(ngrp):
                    v = plsc.load_gather(buf, [rowv[cg], colv]) * scales[cg]
                    plsc.store_scatter(tbuf, [dtv, drv, rowv[cg]], v)

            plsc.parallel_loop(0, _D, unroll=4)(p2)

        for j in range(_NBUF - 1):
            start_gather(j, j)

        def outer(t, carry):
            for u in range(_NBUF):
                h = t * _NBUF + u
                j = u
                jj = (j + _NBUF - 1) % _NBUF
                wait_gather(j)
                compute(j)

                @pl.when(h >= 1)
                def _drain():
                    wait_out(jj)

                start_out(h, j)

                @pl.when(h + _NBUF - 1 < n_chunks)
                def _prefetch():
                    start_gather(h + _NBUF - 1, jj)

            return carry

        lax.fori_loop(0, n_chunks // _NBUF, outer, 0)
        wait_out((n_chunks - 1) % _NBUF)

    return k


def kernel(indices, node_emb):
    bsz, fields = indices.shape
    n_rows = bsz * fields
    assert n_rows % (_NW * _CHUNK) == 0, n_rows
    assert bsz % _CHUNK == 0
    n_chunks = n_rows // (_NW * _CHUNK)
    bpf = bsz // _CHUNK
    idx3 = indices.T.reshape(_NW, n_chunks, _CHUNK)
    out5 = _make_kernel(n_chunks, bpf, fields)(idx3, node_emb)
    return out5.transpose(2, 4, 0, 1, 3).reshape(bsz, fields, _D)
